# baseline (speedup 1.0000x reference)
import dataclasses
import functools

import jax
import jax.numpy as jnp
import numpy as np
from jax import lax
from jax.experimental import pallas as pl
from jax.experimental.pallas import tpu as pltpu
from jax.experimental.pallas import tpu_sc as plsc

N, S, D = 10000, 16, 256
L = 16
CHUNK = 16
NCHUNKS = N // CHUNK
NW = 32
BLK = 1000
NB = N // BLK


def _scalars_body(h0_ref, h1_ref, h3_ref, w0_ref, w1_ref, w3_ref, *out_refs):
    p0 = jnp.dot(h0_ref[...], w0_ref[...], preferred_element_type=jnp.float32)
    p1 = jnp.dot(h1_ref[...], w1_ref[...], preferred_element_type=jnp.float32)
    p3 = jnp.dot(h3_ref[...], w3_ref[...], preferred_element_type=jnp.float32)
    outs = (
        [p0[:, k * L:(k + 1) * L] for k in range(4)]
        + [p1[:, k * L:(k + 1) * L] for k in range(5)]
        + [p3[:, 0:L]]
    )
    for ref, val in zip(out_refs, outs):
        ref[...] = val


def _compute_scalars(h0, h1, h3, w0, w1, w3):
    out_shapes = [jax.ShapeDtypeStruct((N, L), jnp.float32)] * 10
    full = lambda shape: pl.BlockSpec(shape, lambda i: (0, 0))
    return pl.pallas_call(
        _scalars_body,
        grid=(NB,),
        in_specs=[
            pl.BlockSpec((BLK, D), lambda i: (i, 0)),
            pl.BlockSpec((BLK, D), lambda i: (i, 0)),
            pl.BlockSpec((BLK, D), lambda i: (i, 0)),
            full((D, 4 * L)),
            full((D, 5 * L)),
            full((D, 1 * L)),
        ],
        out_specs=[pl.BlockSpec((BLK, L), lambda i: (i, 0))] * 10,
        out_shape=out_shapes,
    )(h0, h1, h3, w0, w1, w3)


def _sc_compiler_params():
    cp = pltpu.CompilerParams()
    if "needs_layout_passes" in pltpu.CompilerParams.__dataclass_fields__:
        cp = dataclasses.replace(cp, needs_layout_passes=False)
    return cp


_SIGMA = np.empty((D,), np.int32)
for _g in range(D // 32):
    for _m in range(16):
        _SIGMA[32 * _g + 2 * _m] = 32 * _g + _m
        _SIGMA[32 * _g + 2 * _m + 1] = 32 * _g + 16 + _m


def _intra_sc(h_src, nei_flat, s_flat, r_flat):
    mesh = plsc.VectorSubcoreMesh(core_axis_name="c", subcore_axis_name="s")

    CS = CHUNK * S
    NT = -(-NCHUNKS // NW)
    assert NT % 2 == 0

    @functools.partial(
        pl.kernel,
        out_type=jax.ShapeDtypeStruct((N, D), jnp.float32),
        mesh=mesh,
        compiler_params=_sc_compiler_params(),
        scratch_types=[
            pltpu.VMEM((CS,), jnp.int32),
            pltpu.VMEM((CS,), jnp.int32),
            pltpu.VMEM((CS, D // 2), jnp.int32),
            pltpu.VMEM((CS, D // 2), jnp.int32),
            pltpu.VMEM((CHUNK, D), jnp.float32),
            pltpu.VMEM((CHUNK, D), jnp.float32),
            pltpu.VMEM((N,), jnp.float32),
            pltpu.VMEM((N,), jnp.float32),
            pltpu.SemaphoreType.DMA,
            pltpu.SemaphoreType.DMA,
            pltpu.SemaphoreType.DMA,
            pltpu.SemaphoreType.DMA,
            pltpu.SemaphoreType.DMA,
            pltpu.SemaphoreType.DMA,
        ],
    )
    def kern(h_hbm, nei_hbm, s_hbm, r_hbm, out_hbm,
             idx0, idx1, emb0, emb1, o0, o1, s_full, r_full,
             si0, si1, sg0, sg1, so0, so1):
        wid = lax.axis_index("s") * 2 + lax.axis_index("c")
        pltpu.sync_copy(s_hbm, s_full)
        pltpu.sync_copy(r_hbm, r_full)

        def cid(t):
            return wid + t * NW

        def start_idx(t, idx_v, sem):
            c = cid(t)

            @pl.when(c < NCHUNKS)
            def _():
                pltpu.async_copy(nei_hbm.at[pl.ds(c * CS, CS)], idx_v, sem)

        def wait_idx_start_gather(t, idx_v, sem_i, emb_v, sem_g):
            c = cid(t)

            @pl.when(c < NCHUNKS)
            def _():
                pltpu.make_async_copy(
                    nei_hbm.at[pl.ds(c * CS, CS)], idx_v, sem_i).wait()
                half = CS // 2
                pltpu.async_copy(h_hbm.at[idx_v.at[pl.ds(0, half)]],
                                 emb_v.at[pl.ds(0, half)], sem_g)
                pltpu.async_copy(h_hbm.at[idx_v.at[pl.ds(half, half)]],
                                 emb_v.at[pl.ds(half, half)], sem_g)

        def out_wait(t, o_v, sem_o):
            c = cid(t)

            @pl.when((t >= 0) & (c < NCHUNKS))
            def _():
                pltpu.make_async_copy(
                    o_v, out_hbm.at[pl.ds(c * CHUNK, CHUNK)], sem_o).wait()

        def wait_gather(t, idx_v, emb_v, sem_g):
            c = cid(t)

            @pl.when(c < NCHUNKS)
            def _():
                half = CS // 2
                pltpu.make_async_copy(h_hbm.at[idx_v.at[pl.ds(0, half)]],
                                      emb_v.at[pl.ds(0, half)], sem_g).wait()
                pltpu.make_async_copy(h_hbm.at[idx_v.at[pl.ds(half, half)]],
                                      emb_v.at[pl.ds(half, half)], sem_g).wait()

        def compute(t, idx_v, emb_v, o_v, sem_o):
            c = cid(t)

            @pl.when(c < NCHUNKS)
            def _():
                base = c * CHUNK

                @plsc.parallel_loop(0, CHUNK, unroll=4)
                def _node(i):
                    idx16 = idx_v[pl.ds(i * S, S)]
                    sv = plsc.load_gather(s_full, [idx16])
                    rv = plsc.load_gather(
                        r_full, [jnp.broadcast_to(base + i, (S,))])
                    x = rv + sv
                    x = jnp.maximum(x, 0.01 * x)
                    e = jnp.exp(x - jnp.max(x))
                    w = e / jnp.broadcast_to(jnp.sum(e), (S,))
                    ws = [jnp.broadcast_to(w[j], (S,)) for j in range(S)]
                    for g in range(D // 32):
                        pa, pb = [], []
                        for j in range(S):
                            y = plsc.bitcast(
                                emb_v[i * S + j, pl.ds(g * L, L)],
                                jnp.bfloat16)
                            a, b = plsc.unpack(
                                y, format=plsc.PackFormat.INTERLEAVED)
                            pa.append(ws[j] * a)
                            pb.append(ws[j] * b)
                        while len(pa) > 1:
                            pa = [pa[k] + pa[k + 1] for k in range(0, len(pa), 2)]
                            pb = [pb[k] + pb[k + 1] for k in range(0, len(pb), 2)]
                        o_v[i, pl.ds(g * 32, L)] = pa[0]
                        o_v[i, pl.ds(g * 32 + L, L)] = pb[0]

                pltpu.async_copy(o_v, out_hbm.at[pl.ds(base, CHUNK)], sem_o)

        start_idx(0, idx0, si0)
        start_idx(1, idx1, si1)
        wait_idx_start_gather(0, idx0, si0, emb0, sg0)

        @pl.loop(0, NT, step=2)
        def _pipe(t):
            out_wait(t - 2, o0, so0)
            wait_gather(t, idx0, emb0, sg0)
            wait_idx_start_gather(t + 1, idx1, si1, emb1, sg1)
            compute(t, idx0, emb0, o0, so0)
            start_idx(t + 2, idx0, si0)
            out_wait(t - 1, o1, so1)
            wait_gather(t + 1, idx1, emb1, sg1)
            wait_idx_start_gather(t + 2, idx0, si0, emb0, sg0)
            compute(t + 1, idx1, emb1, o1, so1)
            start_idx(t + 3, idx1, si1)

        out_wait(NT - 2, o0, so0)
        out_wait(NT - 1, o1, so1)

    return kern(h_src, nei_flat, s_flat, r_flat)


def _elu(x):
    return jnp.where(x > 0, x, jnp.exp(jnp.minimum(x, 0.0)) - 1.0)


def _sums_body(*refs):
    nt = len(refs) - 3
    e_refs, w_ref, b_ref, out_ref = refs[:nt], refs[nt], refs[nt + 1], refs[nt + 2]
    i = pl.program_id(0)

    @pl.when(i == 0)
    def _():
        out_ref[...] = jnp.zeros_like(out_ref)

    w = w_ref[...]
    b = b_ref[...]
    for t in range(nt):
        x = _elu(e_refs[t][...])
        y = jnp.tanh(
            lax.dot_general(x, w, (((1,), (1,)), ((), ())),
                            preferred_element_type=jnp.float32) + b)
        out_ref[pl.ds(t, 1), :] += jnp.sum(y, axis=0, keepdims=True)


def _type_sums(es, fc_w, fc_b):
    nt = len(es)
    return pl.pallas_call(
        _sums_body,
        grid=(NB,),
        in_specs=[pl.BlockSpec((BLK, D), lambda i: (i, 0))] * nt
        + [pl.BlockSpec((D, D), lambda i: (0, 0)),
           pl.BlockSpec((1, D), lambda i: (0, 0))],
        out_specs=pl.BlockSpec((8, D), lambda i: (0, 0)),
        out_shape=jax.ShapeDtypeStruct((8, D), jnp.float32),
    )(*es, fc_w, fc_b.reshape(1, D))


def _combine_body(*refs):
    nt = len(refs) - 2
    e_refs, beta_ref, out_ref = refs[:nt], refs[nt], refs[nt + 1]
    acc = beta_ref[0:1, 0:1] * _elu(e_refs[0][...])
    for t in range(1, nt):
        acc = acc + beta_ref[0:1, t:t + 1] * _elu(e_refs[t][...])
    out_ref[...] = acc


def _combine(es, beta_pad):
    nt = len(es)
    return pl.pallas_call(
        _combine_body,
        grid=(NB,),
        in_specs=[pl.BlockSpec((BLK, D), lambda i: (i, 0))] * nt
        + [pl.BlockSpec((1, 128), lambda i: (0, 0))],
        out_specs=pl.BlockSpec((BLK, D), lambda i: (i, 0)),
        out_shape=jax.ShapeDtypeStruct((N, D), jnp.float32),
    )(*es, beta_pad)


def kernel(h0, h1, h2, h3, nei_d0, nei_d1, nei_p0, nei_p1, nei_p2,
           att_d0, att_d1, att_p0, att_p1, att_p2,
           fc_d_w, fc_d_b, att_inter_d, fc_p_w, fc_p_b, att_inter_p):
    ones = jnp.ones((1, L), jnp.float32)
    rep = lambda col: col.reshape(D, 1) * ones

    w0 = jnp.concatenate(
        [rep(att_d0[0, D:]), rep(att_p0[0, D:]),
         rep(att_d0[0, :D]), rep(att_d1[0, :D])], axis=1)
    w1 = jnp.concatenate(
        [rep(att_d1[0, D:]), rep(att_p1[0, D:]),
         rep(att_p0[0, :D]), rep(att_p1[0, :D]), rep(att_p2[0, :D])], axis=1)
    w3 = rep(att_p2[0, D:])

    (s_d0, s_p0, r_d0, r_d1,
     s_d1, s_p1, r_p0, r_p1, r_p2,
     s_p2) = _compute_scalars(h0, h1, h3, w0, w1, w3)

    flat = lambda nei: nei.astype(jnp.int32).reshape(-1)
    col = lambda a: a[:, 0]
    sig = jnp.asarray(_SIGMA)
    bf = lambda h: lax.bitcast_convert_type(
        h.astype(jnp.bfloat16)[:, sig].reshape(N, D // 2, 2), jnp.int32)
    h0b, h1b, h3b = bf(h0), bf(h1), bf(h3)
    e_d0 = _intra_sc(h0b, flat(nei_d0), col(s_d0), col(r_d0))
    e_d1 = _intra_sc(h1b, flat(nei_d1), col(s_d1), col(r_d1))
    e_p0 = _intra_sc(h0b, flat(nei_p0), col(s_p0), col(r_p0))
    e_p1 = _intra_sc(h1b, flat(nei_p1), col(s_p1), col(r_p1))
    e_p2 = _intra_sc(h3b, flat(nei_p2), col(s_p2), col(r_p2))

    def side(es, fc_w, fc_b, att_inter):
        nt = len(es)
        sums = _type_sums(es, fc_w, fc_b)
        sp = sums[:nt] / N
        logits = (sp * att_inter).sum(axis=1)
        beta = jax.nn.softmax(logits)
        beta_pad = jnp.zeros((1, 128), jnp.float32).at[0, :nt].set(beta)
        return _combine(es, beta_pad)

    z_d = side([e_d0, e_d1], fc_d_w, fc_d_b, att_inter_d)
    z_p = side([e_p0, e_p1, e_p2], fc_p_w, fc_p_b, att_inter_p)
    return (z_d, z_p)


# speedup vs baseline: 1.1691x; 1.1691x over previous
import dataclasses
import functools

import jax
import jax.numpy as jnp
import numpy as np
from jax import lax
from jax.experimental import pallas as pl
from jax.experimental.pallas import tpu as pltpu
from jax.experimental.pallas import tpu_sc as plsc

N, S, D = 10000, 16, 256
L = 16
CHUNK = 16
NCHUNKS = N // CHUNK
NW = 32
BLK = 1000
NB = N // BLK


def _scalars_body(h0_ref, h1_ref, h3_ref, w0_ref, w1_ref, w3_ref, *out_refs):
    p0 = jnp.dot(h0_ref[...], w0_ref[...], preferred_element_type=jnp.float32)
    p1 = jnp.dot(h1_ref[...], w1_ref[...], preferred_element_type=jnp.float32)
    p3 = jnp.dot(h3_ref[...], w3_ref[...], preferred_element_type=jnp.float32)
    outs = (
        [p0[:, k * L:(k + 1) * L] for k in range(4)]
        + [p1[:, k * L:(k + 1) * L] for k in range(5)]
        + [p3[:, 0:L]]
    )
    for ref, val in zip(out_refs, outs):
        ref[...] = val


def _compute_scalars(h0, h1, h3, w0, w1, w3):
    out_shapes = [jax.ShapeDtypeStruct((N, L), jnp.float32)] * 10
    full = lambda shape: pl.BlockSpec(shape, lambda i: (0, 0))
    return pl.pallas_call(
        _scalars_body,
        grid=(NB,),
        in_specs=[
            pl.BlockSpec((BLK, D), lambda i: (i, 0)),
            pl.BlockSpec((BLK, D), lambda i: (i, 0)),
            pl.BlockSpec((BLK, D), lambda i: (i, 0)),
            full((D, 4 * L)),
            full((D, 5 * L)),
            full((D, 1 * L)),
        ],
        out_specs=[pl.BlockSpec((BLK, L), lambda i: (i, 0))] * 10,
        out_shape=out_shapes,
    )(h0, h1, h3, w0, w1, w3)


def _sc_compiler_params():
    cp = pltpu.CompilerParams()
    if "needs_layout_passes" in pltpu.CompilerParams.__dataclass_fields__:
        cp = dataclasses.replace(cp, needs_layout_passes=False)
    return cp


_SIGMA = np.empty((D,), np.int32)
for _g in range(D // 32):
    for _m in range(16):
        _SIGMA[32 * _g + 2 * _m] = 32 * _g + _m
        _SIGMA[32 * _g + 2 * _m + 1] = 32 * _g + 16 + _m


def _intra_sc(h_src, nei_flat, s_flat, r_flat):
    mesh = plsc.VectorSubcoreMesh(core_axis_name="c", subcore_axis_name="s")

    CS = CHUNK * S
    NT = -(-NCHUNKS // NW)
    assert NT % 2 == 0

    @functools.partial(
        pl.kernel,
        out_type=jax.ShapeDtypeStruct((N, D), jnp.float32),
        mesh=mesh,
        compiler_params=_sc_compiler_params(),
        scratch_types=[
            pltpu.VMEM((CS,), jnp.int32),
            pltpu.VMEM((CS,), jnp.int32),
            pltpu.VMEM((CS, D // 2), jnp.int32),
            pltpu.VMEM((CS, D // 2), jnp.int32),
            pltpu.VMEM((CHUNK, D), jnp.float32),
            pltpu.VMEM((CHUNK, D), jnp.float32),
            pltpu.VMEM((N,), jnp.float32),
            pltpu.VMEM((N,), jnp.float32),
            pltpu.SemaphoreType.DMA,
            pltpu.SemaphoreType.DMA,
            pltpu.SemaphoreType.DMA,
            pltpu.SemaphoreType.DMA,
            pltpu.SemaphoreType.DMA,
            pltpu.SemaphoreType.DMA,
        ],
    )
    def kern(h_hbm, nei_hbm, s_hbm, r_hbm, out_hbm,
             idx0, idx1, emb0, emb1, o0, o1, s_full, r_full,
             si0, si1, sg0, sg1, so0, so1):
        wid = lax.axis_index("s") * 2 + lax.axis_index("c")
        pltpu.sync_copy(s_hbm, s_full)
        pltpu.sync_copy(r_hbm, r_full)

        def cid(t):
            return wid + t * NW

        def start_idx(t, idx_v, sem):
            c = cid(t)

            @pl.when(c < NCHUNKS)
            def _():
                pltpu.async_copy(nei_hbm.at[pl.ds(c * CS, CS)], idx_v, sem)

        def wait_idx_start_gather(t, idx_v, sem_i, emb_v, sem_g):
            c = cid(t)

            @pl.when(c < NCHUNKS)
            def _():
                pltpu.make_async_copy(
                    nei_hbm.at[pl.ds(c * CS, CS)], idx_v, sem_i).wait()
                half = CS // 2
                pltpu.async_copy(h_hbm.at[idx_v.at[pl.ds(0, half)]],
                                 emb_v.at[pl.ds(0, half)], sem_g)
                pltpu.async_copy(h_hbm.at[idx_v.at[pl.ds(half, half)]],
                                 emb_v.at[pl.ds(half, half)], sem_g)

        def out_wait(t, o_v, sem_o):
            c = cid(t)

            @pl.when((t >= 0) & (c < NCHUNKS))
            def _():
                pltpu.make_async_copy(
                    o_v, out_hbm.at[pl.ds(c * CHUNK, CHUNK)], sem_o).wait()

        def wait_gather(t, idx_v, emb_v, sem_g):
            c = cid(t)

            @pl.when(c < NCHUNKS)
            def _():
                half = CS // 2
                pltpu.make_async_copy(h_hbm.at[idx_v.at[pl.ds(0, half)]],
                                      emb_v.at[pl.ds(0, half)], sem_g).wait()
                pltpu.make_async_copy(h_hbm.at[idx_v.at[pl.ds(half, half)]],
                                      emb_v.at[pl.ds(half, half)], sem_g).wait()

        def compute(t, idx_v, emb_v, o_v, sem_o):
            c = cid(t)

            @pl.when(c < NCHUNKS)
            def _():
                base = c * CHUNK

                @plsc.parallel_loop(0, CHUNK, unroll=2)
                def _node(i):
                    idx16 = idx_v[pl.ds(i * S, S)]
                    sv = plsc.load_gather(s_full, [idx16])
                    rv = plsc.load_gather(
                        r_full, [jnp.broadcast_to(base + i, (S,))])
                    x = rv + sv
                    x = jnp.maximum(x, 0.01 * x)
                    e = jnp.exp(x - jnp.max(x))
                    w = e / jnp.broadcast_to(jnp.sum(e), (S,))
                    ws = [jnp.broadcast_to(w[j], (S,)) for j in range(S)]
                    wsb = [plsc.pack(wf, wf, format=plsc.PackFormat.INTERLEAVED)
                           for wf in ws]
                    for g in range(D // 32):
                        ps = []
                        for j in range(S):
                            y = plsc.bitcast(
                                emb_v[i * S + j, pl.ds(g * L, L)],
                                jnp.bfloat16)
                            ps.append(wsb[j] * y)
                        while len(ps) > 1:
                            ps = [ps[k] + ps[k + 1] for k in range(0, len(ps), 2)]
                        a, b = plsc.unpack(
                            ps[0], format=plsc.PackFormat.INTERLEAVED)
                        o_v[i, pl.ds(g * 32, L)] = a
                        o_v[i, pl.ds(g * 32 + L, L)] = b

                pltpu.async_copy(o_v, out_hbm.at[pl.ds(base, CHUNK)], sem_o)

        start_idx(0, idx0, si0)
        start_idx(1, idx1, si1)
        wait_idx_start_gather(0, idx0, si0, emb0, sg0)

        @pl.loop(0, NT, step=2)
        def _pipe(t):
            out_wait(t - 2, o0, so0)
            wait_gather(t, idx0, emb0, sg0)
            wait_idx_start_gather(t + 1, idx1, si1, emb1, sg1)
            compute(t, idx0, emb0, o0, so0)
            start_idx(t + 2, idx0, si0)
            out_wait(t - 1, o1, so1)
            wait_gather(t + 1, idx1, emb1, sg1)
            wait_idx_start_gather(t + 2, idx0, si0, emb0, sg0)
            compute(t + 1, idx1, emb1, o1, so1)
            start_idx(t + 3, idx1, si1)

        out_wait(NT - 2, o0, so0)
        out_wait(NT - 1, o1, so1)

    return kern(h_src, nei_flat, s_flat, r_flat)


def _elu(x):
    return jnp.where(x > 0, x, jnp.exp(jnp.minimum(x, 0.0)) - 1.0)


def _sums_body(*refs):
    nt = len(refs) - 3
    e_refs, w_ref, b_ref, out_ref = refs[:nt], refs[nt], refs[nt + 1], refs[nt + 2]
    i = pl.program_id(0)

    @pl.when(i == 0)
    def _():
        out_ref[...] = jnp.zeros_like(out_ref)

    w = w_ref[...]
    b = b_ref[...]
    for t in range(nt):
        x = _elu(e_refs[t][...])
        y = jnp.tanh(
            lax.dot_general(x, w, (((1,), (1,)), ((), ())),
                            preferred_element_type=jnp.float32) + b)
        out_ref[pl.ds(t, 1), :] += jnp.sum(y, axis=0, keepdims=True)


def _type_sums(es, fc_w, fc_b):
    nt = len(es)
    return pl.pallas_call(
        _sums_body,
        grid=(NB,),
        in_specs=[pl.BlockSpec((BLK, D), lambda i: (i, 0))] * nt
        + [pl.BlockSpec((D, D), lambda i: (0, 0)),
           pl.BlockSpec((1, D), lambda i: (0, 0))],
        out_specs=pl.BlockSpec((8, D), lambda i: (0, 0)),
        out_shape=jax.ShapeDtypeStruct((8, D), jnp.float32),
    )(*es, fc_w, fc_b.reshape(1, D))


def _combine_body(*refs):
    nt = len(refs) - 2
    e_refs, beta_ref, out_ref = refs[:nt], refs[nt], refs[nt + 1]
    acc = beta_ref[0:1, 0:1] * _elu(e_refs[0][...])
    for t in range(1, nt):
        acc = acc + beta_ref[0:1, t:t + 1] * _elu(e_refs[t][...])
    out_ref[...] = acc


def _combine(es, beta_pad):
    nt = len(es)
    return pl.pallas_call(
        _combine_body,
        grid=(NB,),
        in_specs=[pl.BlockSpec((BLK, D), lambda i: (i, 0))] * nt
        + [pl.BlockSpec((1, 128), lambda i: (0, 0))],
        out_specs=pl.BlockSpec((BLK, D), lambda i: (i, 0)),
        out_shape=jax.ShapeDtypeStruct((N, D), jnp.float32),
    )(*es, beta_pad)


def kernel(h0, h1, h2, h3, nei_d0, nei_d1, nei_p0, nei_p1, nei_p2,
           att_d0, att_d1, att_p0, att_p1, att_p2,
           fc_d_w, fc_d_b, att_inter_d, fc_p_w, fc_p_b, att_inter_p):
    ones = jnp.ones((1, L), jnp.float32)
    rep = lambda col: col.reshape(D, 1) * ones

    w0 = jnp.concatenate(
        [rep(att_d0[0, D:]), rep(att_p0[0, D:]),
         rep(att_d0[0, :D]), rep(att_d1[0, :D])], axis=1)
    w1 = jnp.concatenate(
        [rep(att_d1[0, D:]), rep(att_p1[0, D:]),
         rep(att_p0[0, :D]), rep(att_p1[0, :D]), rep(att_p2[0, :D])], axis=1)
    w3 = rep(att_p2[0, D:])

    (s_d0, s_p0, r_d0, r_d1,
     s_d1, s_p1, r_p0, r_p1, r_p2,
     s_p2) = _compute_scalars(h0, h1, h3, w0, w1, w3)

    flat = lambda nei: nei.astype(jnp.int32).reshape(-1)
    col = lambda a: a[:, 0]
    sig = jnp.asarray(_SIGMA)
    bf = lambda h: lax.bitcast_convert_type(
        h.astype(jnp.bfloat16)[:, sig].reshape(N, D // 2, 2), jnp.int32)
    h0b, h1b, h3b = bf(h0), bf(h1), bf(h3)
    e_d0 = _intra_sc(h0b, flat(nei_d0), col(s_d0), col(r_d0))
    e_d1 = _intra_sc(h1b, flat(nei_d1), col(s_d1), col(r_d1))
    e_p0 = _intra_sc(h0b, flat(nei_p0), col(s_p0), col(r_p0))
    e_p1 = _intra_sc(h1b, flat(nei_p1), col(s_p1), col(r_p1))
    e_p2 = _intra_sc(h3b, flat(nei_p2), col(s_p2), col(r_p2))

    def side(es, fc_w, fc_b, att_inter):
        nt = len(es)
        sums = _type_sums(es, fc_w, fc_b)
        sp = sums[:nt] / N
        logits = (sp * att_inter).sum(axis=1)
        beta = jax.nn.softmax(logits)
        beta_pad = jnp.zeros((1, 128), jnp.float32).at[0, :nt].set(beta)
        return _combine(es, beta_pad)

    z_d = side([e_d0, e_d1], fc_d_w, fc_d_b, att_inter_d)
    z_p = side([e_p0, e_p1, e_p2], fc_p_w, fc_p_b, att_inter_p)
    return (z_d, z_p)
